# baseline (speedup 1.0000x reference)
import functools

import jax
import jax.numpy as jnp
from jax import lax
from jax.experimental import pallas as pl
from jax.experimental.pallas import tpu as pltpu, tpu_sc as plsc

N_P = 10000
E_P = 320000
N_L = 10000
E_L = 160000
D = 128

NC = 2
NS = 16
NW = NC * NS
RPT = D // NW
CHUNK = 2000
WCHUNK = 3200

_mesh = plsc.VectorSubcoreMesh(core_axis_name="c", subcore_axis_name="s")


def _wid():
    return lax.axis_index("s") * NC + lax.axis_index("c")


def _round_bf16(x):
    u = lax.bitcast_convert_type(x, jnp.uint32)
    u = u + jnp.uint32(0x7FFF) + ((u >> jnp.uint32(16)) & jnp.uint32(1))
    u = u & jnp.uint32(0xFFFF0000)
    return lax.bitcast_convert_type(u, jnp.float32)


def _dot(a, b, dims):
    return lax.dot_general(_round_bf16(a), _round_bf16(b),
                           (dims, ((), ())),
                           preferred_element_type=jnp.float32)


def _tc_protein_pre(x_ref, wg_ref, asrc_ref, adst_ref, ht_ref, sd_ref):
    ht = _dot(wg_ref[...], x_ref[...], ((0,), (1,)))
    ht_ref[...] = ht
    s = _dot(asrc_ref[...], ht, ((1,), (0,)))
    d = _dot(adst_ref[...], ht, ((1,), (0,)))
    sd_ref[...] = jnp.concatenate([s, d], axis=0)


def _tc_ligand_pre(x_ref, wa_ref, ba_ref, w1_ref, h0_ref, hw_ref):
    h0 = jnp.maximum(_dot(wa_ref[...], x_ref[...], ((0,), (1,)))
                     + ba_ref[...], 0.0)
    h0_ref[...] = h0
    hw_ref[...] = _dot(w1_ref[...], h0, ((0,), (0,)))


def _tc_econtrib(attr_ref, wb_ref, bb_ref, w2_ref, bm_ref, ec_ref):
    emb = jnp.maximum(_dot(wb_ref[...], attr_ref[...], ((0,), (1,)))
                      + bb_ref[...], 0.0)
    ec_ref[...] = _dot(w2_ref[...], emb, ((0,), (0,))) + bm_ref[...]


def _tc_update(h_ref, agg_ref, u1_ref, u2_ref, bu_ref, w1_ref,
               hn_ref, hw_ref):
    hn = jnp.maximum(_dot(u1_ref[...], h_ref[...], ((0,), (0,)))
                     + _dot(u2_ref[...], agg_ref[...], ((0,), (0,)))
                     + bu_ref[...], 0.0)
    hn_ref[...] = hn
    hw_ref[...] = _dot(w1_ref[...], hn, ((0,), (0,)))


def _tc_final(acc_ref, dp_ref, h3_ref, wi1_ref, wi2_ref, bi_ref,
              wkd_ref, bkd_ref, out_ref):
    denom = jnp.sum(dp_ref[...], axis=0, keepdims=True)
    p_nodes = jnp.maximum(acc_ref[...] / (denom + 1e-16), 0.0)
    p = jnp.sum(p_nodes, axis=1, keepdims=True) * (1.0 / N_P)
    l = jnp.sum(h3_ref[...], axis=1, keepdims=True) * (1.0 / N_L)
    inter = jnp.maximum(_dot(wi1_ref[...], p, ((0,), (0,)))
                        + _dot(wi2_ref[...], l, ((0,), (0,)))
                        + bi_ref[...], 0.0)
    out_ref[...] = _dot(wkd_ref[...], inter, ((0,), (0,))) + bkd_ref[...]


def _sc_gat_scalar(s_hbm, d_hbm, src_hbm, dst_hbm, ex_hbm, dp_hbm,
                   s_v, d_v, src_v, dst_v, ex_v, dnm_v):
    wid = _wid()
    ept = E_P // NW
    base = wid * ept
    pltpu.sync_copy(s_hbm, s_v)
    pltpu.sync_copy(d_hbm, d_v)

    def zero_body(i, c):
        dnm_v[pl.ds(i * 16, 16)] = jnp.zeros((16,), jnp.float32)
        return c
    lax.fori_loop(0, N_P // 16, zero_body, 0)

    def chunk_body(g, c):
        off = base + g * CHUNK
        pltpu.sync_copy(src_hbm.at[pl.ds(off, CHUNK)], src_v)
        pltpu.sync_copy(dst_hbm.at[pl.ds(off, CHUNK)], dst_v)

        def body(j, c2):
            sl = pl.ds(j * 16, 16)
            sidx = src_v[sl]
            didx = dst_v[sl]
            sv = plsc.load_gather(s_v, [sidx])
            dv = plsc.load_gather(d_v, [didx])
            e = sv + dv
            e = jnp.maximum(e, 0.2 * e)
            ex = jnp.exp(e)
            ex_v[sl] = ex
            plsc.addupdate_scatter(dnm_v, [didx], ex)
            return c2
        lax.fori_loop(0, CHUNK // 16, body, 0)
        pltpu.sync_copy(ex_v, ex_hbm.at[pl.ds(off, CHUNK)])
        return c
    lax.fori_loop(0, ept // CHUNK, chunk_body, 0)
    pltpu.sync_copy(dnm_v, dp_hbm.at[pl.ds(wid * N_P, N_P)])


def _sc_edge_pass(n_edges, n_nodes, weighted):
    C = WCHUNK
    nch = n_edges // C
    assert nch % 2 == 0

    def body(tab_hbm, src_hbm, dst_hbm, w_hbm, zeros_hbm, acc_hbm,
             tab_v, acc_v, src_v, dst_v, w_v, sem):
        wid = _wid()
        r0 = wid * RPT
        pltpu.sync_copy(tab_hbm.at[pl.ds(r0 * n_nodes, RPT * n_nodes)],
                        tab_v)
        pltpu.sync_copy(zeros_hbm, acc_v)

        def dmas(g, half):
            off = g * C
            hoff = half * C
            yield src_hbm.at[pl.ds(off, C)], src_v.at[pl.ds(hoff, C)]
            yield dst_hbm.at[pl.ds(off, C)], dst_v.at[pl.ds(hoff, C)]
            if weighted:
                yield w_hbm.at[pl.ds(off, C)], w_v.at[pl.ds(hoff, C)]
            else:
                for r in range(RPT):
                    yield (w_hbm.at[pl.ds((r0 + r) * n_edges + off, C)],
                           w_v.at[pl.ds((half * RPT + r) * C, C)])

        def start(g, half):
            for s, dst in dmas(g, half):
                pltpu.async_copy(s, dst, sem.at[half])

        def drain(g, half):
            for s, dst in dmas(g, half):
                pltpu.make_async_copy(s, dst, sem.at[half]).wait()

        def compute(half):
            hoff = half * C

            def body_j(j, c2):
                sl = pl.ds(hoff + j * 16, 16)
                sidx = src_v[sl]
                didx = dst_v[sl]
                if weighted:
                    wv = w_v[sl]
                for r in range(RPT):
                    g16 = plsc.load_gather(tab_v, [sidx + (r * n_nodes)])
                    if weighted:
                        val = g16 * wv
                    else:
                        val = jnp.maximum(
                            g16 + w_v[pl.ds((half * RPT + r) * C + j * 16,
                                            16)], 0.0)
                    plsc.addupdate_scatter(
                        acc_v, [didx + (r * n_nodes)], val)
                return c2
            lax.fori_loop(0, C // 16, body_j, 0, unroll=4)

        start(0, 0)

        def pair_body(p, c):
            g0 = 2 * p
            start(g0 + 1, 1)
            drain(g0, 0)
            compute(0)

            @pl.when(g0 + 2 < nch)
            def _():
                start(g0 + 2, 0)
            drain(g0 + 1, 1)
            compute(1)
            return c
        lax.fori_loop(0, nch // 2, pair_body, 0)
        pltpu.sync_copy(acc_v,
                        acc_hbm.at[pl.ds(r0 * n_nodes, RPT * n_nodes)])
    return body


def _make_sc_gat_scalar():
    return pl.kernel(
        _sc_gat_scalar,
        out_type=[jax.ShapeDtypeStruct((E_P,), jnp.float32),
                  jax.ShapeDtypeStruct((NW * N_P,), jnp.float32)],
        mesh=_mesh,
        compiler_params=pltpu.CompilerParams(needs_layout_passes=False),
        scratch_types=[
            pltpu.VMEM((N_P,), jnp.float32),
            pltpu.VMEM((N_P,), jnp.float32),
            pltpu.VMEM((CHUNK,), jnp.int32),
            pltpu.VMEM((CHUNK,), jnp.int32),
            pltpu.VMEM((CHUNK,), jnp.float32),
            pltpu.VMEM((N_P,), jnp.float32),
        ],
    )


def _make_sc_edge_pass(n_edges, n_nodes, weighted):
    if weighted:
        w_scratch = pltpu.VMEM((2 * WCHUNK,), jnp.float32)
    else:
        w_scratch = pltpu.VMEM((2 * RPT * WCHUNK,), jnp.float32)
    return pl.kernel(
        _sc_edge_pass(n_edges, n_nodes, weighted),
        out_type=jax.ShapeDtypeStruct((D * n_nodes,), jnp.float32),
        mesh=_mesh,
        compiler_params=pltpu.CompilerParams(needs_layout_passes=False),
        scratch_types=[
            pltpu.VMEM((RPT * n_nodes,), jnp.float32),
            pltpu.VMEM((RPT * n_nodes,), jnp.float32),
            pltpu.VMEM((2 * WCHUNK,), jnp.int32),
            pltpu.VMEM((2 * WCHUNK,), jnp.int32),
            w_scratch,
            pltpu.SemaphoreType.DMA((2,)),
        ],
    )


def kernel(protein_x, protein_edge_index, ligand_x, ligand_edge_index,
           ligand_edge_attr, W_atom, b_atom, W_bond, b_bond, W_gat,
           a_src, a_dst, W_msg, b_msg, W_upd, b_upd, W_int, b_int,
           W_kd, b_kd):
    f32 = jnp.float32
    pe_src = protein_edge_index[0]
    pe_dst = protein_edge_index[1]
    le_src = ligand_edge_index[0]
    le_dst = ligand_edge_index[1]

    ht, sd = pl.pallas_call(
        _tc_protein_pre,
        out_shape=[jax.ShapeDtypeStruct((D, N_P), f32),
                   jax.ShapeDtypeStruct((2, N_P), f32)],
    )(protein_x, W_gat, a_src.reshape(1, D), a_dst.reshape(1, D))
    s = sd[0]
    d_vec = sd[1]

    ex_arr, dparts = _make_sc_gat_scalar()(s, d_vec, pe_src, pe_dst)
    dparts = dparts.reshape(NW, N_P)

    zeros_tile = jnp.zeros((RPT * N_P,), f32)
    acc = _make_sc_edge_pass(E_P, N_P, True)(
        ht.reshape(-1), pe_src, pe_dst, ex_arr, zeros_tile)
    acc = acc.reshape(D, N_P)

    W1 = W_msg[:D]
    W2 = W_msg[D:]
    h0, hw = pl.pallas_call(
        _tc_ligand_pre,
        out_shape=[jax.ShapeDtypeStruct((D, N_L), f32),
                   jax.ShapeDtypeStruct((D, N_L), f32)],
    )(ligand_x, W_atom, b_atom.reshape(D, 1), W1)

    EB = E_L // 10
    ec = pl.pallas_call(
        _tc_econtrib,
        grid=(10,),
        in_specs=[
            pl.BlockSpec((EB, 16), lambda i: (i, 0)),
            pl.BlockSpec((16, D), lambda i: (0, 0)),
            pl.BlockSpec((D, 1), lambda i: (0, 0)),
            pl.BlockSpec((D, D), lambda i: (0, 0)),
            pl.BlockSpec((D, 1), lambda i: (0, 0)),
        ],
        out_specs=pl.BlockSpec((D, EB), lambda i: (0, i)),
        out_shape=jax.ShapeDtypeStruct((D, E_L), f32),
    )(ligand_edge_attr, W_bond, b_bond.reshape(D, 1), W2,
      b_msg.reshape(D, 1))

    mpnn_pass = _make_sc_edge_pass(E_L, N_L, False)
    ec_flat = ec.reshape(-1)
    upd = functools.partial(
        pl.pallas_call, _tc_update,
        out_shape=[jax.ShapeDtypeStruct((D, N_L), f32),
                   jax.ShapeDtypeStruct((D, N_L), f32)])
    h = h0
    for _ in range(3):
        agg = mpnn_pass(hw.reshape(-1), le_src, le_dst, ec_flat,
                        zeros_tile)
        h, hw = upd()(h, agg.reshape(D, N_L), W_upd[:D], W_upd[D:],
                      b_upd.reshape(D, 1), W1)

    kd = pl.pallas_call(
        _tc_final,
        out_shape=jax.ShapeDtypeStruct((1, 1), f32),
    )(acc, dparts, h, W_int[:D], W_int[D:], b_int.reshape(D, 1),
      W_kd, b_kd.reshape(1, 1))
    return kd.reshape(1)


# speedup vs baseline: 2.1379x; 2.1379x over previous
import functools

import jax
import jax.numpy as jnp
from jax import lax
from jax.experimental import pallas as pl
from jax.experimental.pallas import tpu as pltpu, tpu_sc as plsc

N_P = 10000
E_P = 320000
N_L = 10000
E_L = 160000
D = 128

NC = 2
NS = 16
NW = NC * NS
RPT = D // NW
CHUNK = 2000
WCHUNK = 3200

_mesh = plsc.VectorSubcoreMesh(core_axis_name="c", subcore_axis_name="s")


def _wid():
    return lax.axis_index("s") * NC + lax.axis_index("c")


def _round_bf16(x):
    u = lax.bitcast_convert_type(x, jnp.uint32)
    u = u + jnp.uint32(0x7FFF) + ((u >> jnp.uint32(16)) & jnp.uint32(1))
    u = u & jnp.uint32(0xFFFF0000)
    return lax.bitcast_convert_type(u, jnp.float32)


def _dot(a, b, dims):
    return lax.dot_general(_round_bf16(a), _round_bf16(b),
                           (dims, ((), ())),
                           preferred_element_type=jnp.float32)


def _tc_protein_pre(x_ref, wg_ref, asrc_ref, adst_ref, ht_ref, sd_ref):
    ht = _dot(wg_ref[...], x_ref[...], ((0,), (1,)))
    ht_ref[...] = ht
    s = _dot(asrc_ref[...], ht, ((1,), (0,)))
    d = _dot(adst_ref[...], ht, ((1,), (0,)))
    sd_ref[...] = jnp.concatenate([s, d], axis=0)


def _tc_ligand_pre(x_ref, wa_ref, ba_ref, w1_ref, h0_ref, hw_ref):
    h0 = jnp.maximum(_dot(wa_ref[...], x_ref[...], ((0,), (1,)))
                     + ba_ref[...], 0.0)
    h0_ref[...] = h0
    hw_ref[...] = _dot(w1_ref[...], h0, ((0,), (0,)))


def _tc_econtrib(attr_ref, wb_ref, bb_ref, w2_ref, bm_ref, ec_ref):
    emb = jnp.maximum(_dot(wb_ref[...], attr_ref[...], ((0,), (1,)))
                      + bb_ref[...], 0.0)
    ec_ref[...] = _dot(w2_ref[...], emb, ((0,), (0,))) + bm_ref[...]


def _tc_update(h_ref, agg_ref, u1_ref, u2_ref, bu_ref, w1_ref,
               hn_ref, hw_ref):
    hn = jnp.maximum(_dot(u1_ref[...], h_ref[...], ((0,), (0,)))
                     + _dot(u2_ref[...], agg_ref[...], ((0,), (0,)))
                     + bu_ref[...], 0.0)
    hn_ref[...] = hn
    hw_ref[...] = _dot(w1_ref[...], hn, ((0,), (0,)))


def _tc_final(acc_ref, dp_ref, h3_ref, wi1_ref, wi2_ref, bi_ref,
              wkd_ref, bkd_ref, out_ref):
    denom = jnp.sum(dp_ref[...], axis=0, keepdims=True)
    p_nodes = jnp.maximum(acc_ref[...] / (denom + 1e-16), 0.0)
    p = jnp.sum(p_nodes, axis=1, keepdims=True) * (1.0 / N_P)
    l = jnp.sum(h3_ref[...], axis=1, keepdims=True) * (1.0 / N_L)
    inter = jnp.maximum(_dot(wi1_ref[...], p, ((0,), (0,)))
                        + _dot(wi2_ref[...], l, ((0,), (0,)))
                        + bi_ref[...], 0.0)
    out_ref[...] = _dot(wkd_ref[...], inter, ((0,), (0,))) + bkd_ref[...]


def _sc_gat_scalar(s_hbm, d_hbm, src_hbm, dst_hbm, ex_hbm, dp_hbm,
                   s_v, d_v, src_v, dst_v, ex_v, dnm_v):
    wid = _wid()
    ept = E_P // NW
    base = wid * ept
    pltpu.sync_copy(s_hbm, s_v)
    pltpu.sync_copy(d_hbm, d_v)

    def zero_body(i, c):
        dnm_v[pl.ds(i * 16, 16)] = jnp.zeros((16,), jnp.float32)
        return c
    lax.fori_loop(0, N_P // 16, zero_body, 0)

    def chunk_body(g, c):
        off = base + g * CHUNK
        pltpu.sync_copy(src_hbm.at[pl.ds(off, CHUNK)], src_v)
        pltpu.sync_copy(dst_hbm.at[pl.ds(off, CHUNK)], dst_v)

        def body(j, c2):
            sl = pl.ds(j * 16, 16)
            sidx = src_v[sl]
            didx = dst_v[sl]
            sv = plsc.load_gather(s_v, [sidx])
            dv = plsc.load_gather(d_v, [didx])
            e = sv + dv
            e = jnp.maximum(e, 0.2 * e)
            ex = jnp.exp(e)
            ex_v[sl] = ex
            plsc.addupdate_scatter(dnm_v, [didx], ex)
            return c2
        lax.fori_loop(0, CHUNK // 16, body, 0)
        pltpu.sync_copy(ex_v, ex_hbm.at[pl.ds(off, CHUNK)])
        return c
    lax.fori_loop(0, ept // CHUNK, chunk_body, 0)
    pltpu.sync_copy(dnm_v, dp_hbm.at[pl.ds(wid * N_P, N_P)])


def _sc_edge_pass(n_edges, n_nodes, weighted):
    C = WCHUNK
    nch = n_edges // C
    assert nch % 2 == 0

    def body(tab_hbm, src_hbm, dst_hbm, w_hbm, zeros_hbm, acc_hbm,
             tab_v, acc_v, src_v, dst_v, w_v, sem):
        wid = _wid()
        r0 = wid * RPT
        pltpu.sync_copy(tab_hbm.at[pl.ds(r0 * n_nodes, RPT * n_nodes)],
                        tab_v)
        pltpu.sync_copy(zeros_hbm, acc_v)

        def dmas(g, half):
            off = g * C
            hoff = half * C
            yield src_hbm.at[pl.ds(off, C)], src_v.at[pl.ds(hoff, C)]
            yield dst_hbm.at[pl.ds(off, C)], dst_v.at[pl.ds(hoff, C)]
            if weighted:
                yield w_hbm.at[pl.ds(off, C)], w_v.at[pl.ds(hoff, C)]
            else:
                for r in range(RPT):
                    yield (w_hbm.at[pl.ds((r0 + r) * n_edges + off, C)],
                           w_v.at[pl.ds((half * RPT + r) * C, C)])

        def start(g, half):
            for s, dst in dmas(g, half):
                pltpu.async_copy(s, dst, sem.at[half])

        def drain(g, half):
            for s, dst in dmas(g, half):
                pltpu.make_async_copy(s, dst, sem.at[half]).wait()

        def compute(half):
            hoff = half * C

            @plsc.parallel_loop(0, C // 16, 1, unroll=2)
            def body_j(j):
                sl = pl.ds(hoff + j * 16, 16)
                sidx = src_v[sl]
                didx = dst_v[sl]
                if weighted:
                    wv = w_v[sl]
                for r in range(RPT):
                    g16 = plsc.load_gather(tab_v, [sidx + (r * n_nodes)])
                    if weighted:
                        val = g16 * wv
                    else:
                        val = jnp.maximum(
                            g16 + w_v[pl.ds((half * RPT + r) * C + j * 16,
                                            16)], 0.0)
                    plsc.addupdate_scatter(
                        acc_v, [didx + (r * n_nodes)], val)

        start(0, 0)

        def pair_body(p, c):
            g0 = 2 * p
            start(g0 + 1, 1)
            drain(g0, 0)
            compute(0)

            @pl.when(g0 + 2 < nch)
            def _():
                start(g0 + 2, 0)
            drain(g0 + 1, 1)
            compute(1)
            return c
        lax.fori_loop(0, nch // 2, pair_body, 0)
        pltpu.sync_copy(acc_v,
                        acc_hbm.at[pl.ds(r0 * n_nodes, RPT * n_nodes)])
    return body


def _make_sc_gat_scalar():
    return pl.kernel(
        _sc_gat_scalar,
        out_type=[jax.ShapeDtypeStruct((E_P,), jnp.float32),
                  jax.ShapeDtypeStruct((NW * N_P,), jnp.float32)],
        mesh=_mesh,
        compiler_params=pltpu.CompilerParams(needs_layout_passes=False),
        scratch_types=[
            pltpu.VMEM((N_P,), jnp.float32),
            pltpu.VMEM((N_P,), jnp.float32),
            pltpu.VMEM((CHUNK,), jnp.int32),
            pltpu.VMEM((CHUNK,), jnp.int32),
            pltpu.VMEM((CHUNK,), jnp.float32),
            pltpu.VMEM((N_P,), jnp.float32),
        ],
    )


def _make_sc_edge_pass(n_edges, n_nodes, weighted):
    if weighted:
        w_scratch = pltpu.VMEM((2 * WCHUNK,), jnp.float32)
    else:
        w_scratch = pltpu.VMEM((2 * RPT * WCHUNK,), jnp.float32)
    return pl.kernel(
        _sc_edge_pass(n_edges, n_nodes, weighted),
        out_type=jax.ShapeDtypeStruct((D * n_nodes,), jnp.float32),
        mesh=_mesh,
        compiler_params=pltpu.CompilerParams(needs_layout_passes=False),
        scratch_types=[
            pltpu.VMEM((RPT * n_nodes,), jnp.float32),
            pltpu.VMEM((RPT * n_nodes,), jnp.float32),
            pltpu.VMEM((2 * WCHUNK,), jnp.int32),
            pltpu.VMEM((2 * WCHUNK,), jnp.int32),
            w_scratch,
            pltpu.SemaphoreType.DMA((2,)),
        ],
    )


def kernel(protein_x, protein_edge_index, ligand_x, ligand_edge_index,
           ligand_edge_attr, W_atom, b_atom, W_bond, b_bond, W_gat,
           a_src, a_dst, W_msg, b_msg, W_upd, b_upd, W_int, b_int,
           W_kd, b_kd):
    f32 = jnp.float32
    pe_src = protein_edge_index[0]
    pe_dst = protein_edge_index[1]
    le_src = ligand_edge_index[0]
    le_dst = ligand_edge_index[1]

    ht, sd = pl.pallas_call(
        _tc_protein_pre,
        out_shape=[jax.ShapeDtypeStruct((D, N_P), f32),
                   jax.ShapeDtypeStruct((2, N_P), f32)],
    )(protein_x, W_gat, a_src.reshape(1, D), a_dst.reshape(1, D))
    s = sd[0]
    d_vec = sd[1]

    ex_arr, dparts = _make_sc_gat_scalar()(s, d_vec, pe_src, pe_dst)
    dparts = dparts.reshape(NW, N_P)

    zeros_tile = jnp.zeros((RPT * N_P,), f32)
    acc = _make_sc_edge_pass(E_P, N_P, True)(
        ht.reshape(-1), pe_src, pe_dst, ex_arr, zeros_tile)
    acc = acc.reshape(D, N_P)

    W1 = W_msg[:D]
    W2 = W_msg[D:]
    h0, hw = pl.pallas_call(
        _tc_ligand_pre,
        out_shape=[jax.ShapeDtypeStruct((D, N_L), f32),
                   jax.ShapeDtypeStruct((D, N_L), f32)],
    )(ligand_x, W_atom, b_atom.reshape(D, 1), W1)

    EB = E_L // 10
    ec = pl.pallas_call(
        _tc_econtrib,
        grid=(10,),
        in_specs=[
            pl.BlockSpec((EB, 16), lambda i: (i, 0)),
            pl.BlockSpec((16, D), lambda i: (0, 0)),
            pl.BlockSpec((D, 1), lambda i: (0, 0)),
            pl.BlockSpec((D, D), lambda i: (0, 0)),
            pl.BlockSpec((D, 1), lambda i: (0, 0)),
        ],
        out_specs=pl.BlockSpec((D, EB), lambda i: (0, i)),
        out_shape=jax.ShapeDtypeStruct((D, E_L), f32),
    )(ligand_edge_attr, W_bond, b_bond.reshape(D, 1), W2,
      b_msg.reshape(D, 1))

    mpnn_pass = _make_sc_edge_pass(E_L, N_L, False)
    ec_flat = ec.reshape(-1)
    upd = functools.partial(
        pl.pallas_call, _tc_update,
        out_shape=[jax.ShapeDtypeStruct((D, N_L), f32),
                   jax.ShapeDtypeStruct((D, N_L), f32)])
    h = h0
    for _ in range(3):
        agg = mpnn_pass(hw.reshape(-1), le_src, le_dst, ec_flat,
                        zeros_tile)
        h, hw = upd()(h, agg.reshape(D, N_L), W_upd[:D], W_upd[D:],
                      b_upd.reshape(D, 1), W1)

    kd = pl.pallas_call(
        _tc_final,
        out_shape=jax.ShapeDtypeStruct((1, 1), f32),
    )(acc, dparts, h, W_int[:D], W_int[D:], b_int.reshape(D, 1),
      W_kd, b_kd.reshape(1, 1))
    return kd.reshape(1)
